# baseline (speedup 1.0000x reference)
import functools

import jax
import jax.numpy as jnp
from jax import lax
from jax.experimental import pallas as pl
from jax.experimental.pallas import tpu as pltpu
from jax.experimental.pallas import tpu_sc as plsc

N = 10000
D = 128
G = 64
E = 320000

NC = 2
NS = 16
K = 64
C = 157
SEC = 32
NSEC = (C + SEC - 1) // SEC
CPAD = NSEC * SEC
E_PAD = NC * NS * C * K
ZROWS = 632
NP = NS * ZROWS


_MESH = plsc.VectorSubcoreMesh(
    core_axis_name="c", subcore_axis_name="s", num_cores=NC, num_subcores=NS)


@functools.partial(
    pl.kernel,
    out_type=jax.ShapeDtypeStruct((NC, NP, D), jnp.float32),
    mesh=_MESH,
    scratch_types=[
        pltpu.VMEM((2, SEC, K), jnp.int32),
        pltpu.VMEM((2, SEC, K), jnp.int32),
        pltpu.VMEM((K, D), jnp.float32),
        pltpu.VMEM((K, D), jnp.float32),
        pltpu.VMEM((K, D), jnp.float32),
        pltpu.VMEM((K, D), jnp.float32),
        pltpu.VMEM((K, D), jnp.float32),
        pltpu.VMEM_SHARED((NP, D), jnp.float32),
    ] + [pltpu.SemaphoreType.DMA] * 12,
    compiler_params=pltpu.CompilerParams(use_tc_tiling_on_sc=False),
)
def _sc_aggregate(h_hbm, src_hbm, dst_hbm, out_hbm,
                  idxs_v, idxd_v, gbuf0, gbuf1, gbuf2, gbuf3, gbuf4,
                  agg, *sems):
    c = lax.axis_index("c")
    s = lax.axis_index("s")
    isems = sems[10:]
    pltpu.sync_copy(src_hbm.at[c, s, 0], idxs_v.at[0])
    pltpu.sync_copy(dst_hbm.at[c, s, 0], idxd_v.at[0])
    ipend = [
        pltpu.async_copy(src_hbm.at[c, s, 1], idxs_v.at[1], isems[0]),
        pltpu.async_copy(dst_hbm.at[c, s, 1], idxd_v.at[1], isems[1]),
    ]
    z = jnp.zeros((16,), jnp.float32)

    def _zrow(i, carry):
        def _zcol(jj, cc):
            gbuf0[i, pl.ds(jj * 16, 16)] = z
            return cc
        return lax.fori_loop(0, D // 16, _zcol, carry)

    lax.fori_loop(0, K, _zrow, 0)
    for k in range(9):
        pltpu.sync_copy(gbuf0, agg.at[pl.ds(s * ZROWS + k * K, K)])
    pltpu.sync_copy(gbuf0.at[pl.ds(0, ZROWS - 9 * K)],
                    agg.at[pl.ds(s * ZROWS + 9 * K, ZROWS - 9 * K)])
    plsc.subcore_barrier()

    bufs = [gbuf0, gbuf1, gbuf2, gbuf3, gbuf4]
    gsems = sems[:5]
    ssems = sems[5:10]

    def gath(j):
        sec, jj = divmod(j, SEC)
        return pltpu.async_copy(h_hbm.at[idxs_v.at[sec % 2, jj]],
                                bufs[j % 5], gsems[j % 5])

    def scat(j):
        sec, jj = divmod(j, SEC)
        return pltpu.async_copy(bufs[j % 5], agg.at[idxd_v.at[sec % 2, jj]],
                                ssems[j % 5], add=True)

    gp = [gath(0), gath(1), gath(2)]
    sp = [None] * 5
    for j in range(C):
        gp[j % 3].wait()
        sp[j % 5] = scat(j)
        if j >= 2:
            sp[(j - 2) % 5].wait()
        nxt = j + 3
        if nxt < C:
            if nxt % SEC < 3 and nxt // SEC > j // SEC:
                if ipend:
                    ipend[0].wait()
                    ipend[1].wait()
                    ipend = []
            gp[j % 3] = gath(nxt)
        if j % SEC == 2 and j >= SEC and (j // SEC + 1) < NSEC:
            nsec = j // SEC + 1
            ipend = [
                pltpu.async_copy(src_hbm.at[c, s, nsec],
                                 idxs_v.at[nsec % 2], isems[0]),
                pltpu.async_copy(dst_hbm.at[c, s, nsec],
                                 idxd_v.at[nsec % 2], isems[1]),
            ]
    sp[(C - 1) % 5].wait()
    sp[(C - 2) % 5].wait()

    plsc.subcore_barrier()
    pltpu.sync_copy(agg.at[pl.ds(s * ZROWS, ZROWS)],
                    out_hbm.at[c, pl.ds(s * ZROWS, ZROWS)])


RB = 5000
NB = N // RB


def _mlp_body(h_ref, a_ref, b3_ref, w1_ref, b1_ref, w2_ref, b2_ref,
              hn_ref, pooled_ref):
    m = h_ref[...] + a_ref[0] + a_ref[1]
    t = jnp.maximum(
        jnp.dot(m, w1_ref[...], preferred_element_type=jnp.float32)
        + b1_ref[...], 0.0)
    hn = (jnp.dot(t, w2_ref[...], preferred_element_type=jnp.float32)
          + b2_ref[...])
    hn_ref[...] = hn
    b = b3_ref[0, 0, :]
    oh = (b[None, :] == lax.broadcasted_iota(jnp.int32, (G, RB), 0)
          ).astype(jnp.float32)
    contrib = jnp.dot(oh, hn, preferred_element_type=jnp.float32)

    @pl.when(pl.program_id(0) == 0)
    def _():
        pooled_ref[...] = contrib

    @pl.when(pl.program_id(0) != 0)
    def _():
        pooled_ref[...] += contrib


_mlp_call = pl.pallas_call(
    _mlp_body,
    grid=(NB,),
    in_specs=[
        pl.BlockSpec((RB, D), lambda i: (i, 0)),
        pl.BlockSpec((NC, RB, D), lambda i: (0, i, 0)),
        pl.BlockSpec((1, 1, RB), lambda i: (i, 0, 0)),
        pl.BlockSpec((D, D), lambda i: (0, 0)),
        pl.BlockSpec((1, D), lambda i: (0, 0)),
        pl.BlockSpec((D, D), lambda i: (0, 0)),
        pl.BlockSpec((1, D), lambda i: (0, 0)),
    ],
    out_specs=[
        pl.BlockSpec((RB, D), lambda i: (i, 0)),
        pl.BlockSpec((G, D), lambda i: (0, 0)),
    ],
    out_shape=[
        jax.ShapeDtypeStruct((N, D), jnp.float32),
        jax.ShapeDtypeStruct((G, D), jnp.float32),
    ],
)


def _mlp3_body(h_ref, a_ref, b3_ref, w1_ref, b1_ref, w2_ref, b2_ref,
               p0_ref, p1_ref, wp1_ref, bp1_ref, wp2_ref, bp2_ref,
               pooled_ref, out_ref):
    m = h_ref[...] + a_ref[0] + a_ref[1]
    t = jnp.maximum(
        jnp.dot(m, w1_ref[...], preferred_element_type=jnp.float32)
        + b1_ref[...], 0.0)
    hn = (jnp.dot(t, w2_ref[...], preferred_element_type=jnp.float32)
          + b2_ref[...])
    b = b3_ref[0, 0, :]
    oh = (b[None, :] == lax.broadcasted_iota(jnp.int32, (G, RB), 0)
          ).astype(jnp.float32)
    contrib = jnp.dot(oh, hn, preferred_element_type=jnp.float32)

    @pl.when(pl.program_id(0) == 0)
    def _():
        pooled_ref[...] = contrib

    @pl.when(pl.program_id(0) != 0)
    def _():
        pooled_ref[...] += contrib

    @pl.when(pl.program_id(0) == NB - 1)
    def _():
        cat = jnp.concatenate(
            [p0_ref[...], p1_ref[...], pooled_ref[...]], axis=1)
        th = jnp.maximum(
            jnp.dot(cat, wp1_ref[...], preferred_element_type=jnp.float32)
            + bp1_ref[...], 0.0)
        out_ref[...] = (
            jnp.dot(th, wp2_ref[...], preferred_element_type=jnp.float32)
            + bp2_ref[...])


_mlp3_call = pl.pallas_call(
    _mlp3_body,
    grid=(NB,),
    in_specs=[
        pl.BlockSpec((RB, D), lambda i: (i, 0)),
        pl.BlockSpec((NC, RB, D), lambda i: (0, i, 0)),
        pl.BlockSpec((1, 1, RB), lambda i: (i, 0, 0)),
        pl.BlockSpec((D, D), lambda i: (0, 0)),
        pl.BlockSpec((1, D), lambda i: (0, 0)),
        pl.BlockSpec((D, D), lambda i: (0, 0)),
        pl.BlockSpec((1, D), lambda i: (0, 0)),
        pl.BlockSpec((G, D), lambda i: (0, 0)),
        pl.BlockSpec((G, D), lambda i: (0, 0)),
        pl.BlockSpec((3 * D, D), lambda i: (0, 0)),
        pl.BlockSpec((1, D), lambda i: (0, 0)),
        pl.BlockSpec((D, D), lambda i: (0, 0)),
        pl.BlockSpec((1, D), lambda i: (0, 0)),
    ],
    out_specs=[
        pl.BlockSpec((G, D), lambda i: (0, 0)),
        pl.BlockSpec((G, D), lambda i: (0, 0)),
    ],
    out_shape=[
        jax.ShapeDtypeStruct((G, D), jnp.float32),
        jax.ShapeDtypeStruct((G, D), jnp.float32),
    ],
)


def kernel(x, edge_index, batch, w1_0, b1_0, w2_0, b2_0, w1_1, b1_1, w2_1,
           b2_1, w1_2, b1_2, w2_2, b2_2, wp1, bp1, wp2, bp2):
    src = edge_index[0].astype(jnp.int32)
    dst = edge_index[1].astype(jnp.int32)
    npad = E_PAD - E
    pad_i = jnp.arange(npad, dtype=jnp.int32)
    src_p = jnp.concatenate([src, pad_i % 16])
    dst_p = jnp.concatenate([dst, N + (pad_i % 8)])
    src_r = jnp.pad(src_p.reshape(NC, NS, C, K),
                    ((0, 0), (0, 0), (0, CPAD - C), (0, 0))
                    ).reshape(NC, NS, NSEC, SEC, K)
    dst_r = jnp.pad(dst_p.reshape(NC, NS, C, K),
                    ((0, 0), (0, 0), (0, CPAD - C), (0, 0))
                    ).reshape(NC, NS, NSEC, SEC, K)
    batch3 = batch.astype(jnp.int32).reshape(NB, 1, RB)

    h = x
    pooled = []
    for (w1, b1, w2, b2) in [(w1_0, b1_0, w2_0, b2_0),
                             (w1_1, b1_1, w2_1, b2_1)]:
        agg2 = _sc_aggregate(h, src_r, dst_r)
        h, pl_l = _mlp_call(h, agg2, batch3, w1, b1.reshape(1, D),
                            w2, b2.reshape(1, D))
        pooled.append(pl_l)
    agg2 = _sc_aggregate(h, src_r, dst_r)
    _, out = _mlp3_call(h, agg2, batch3, w1_2, b1_2.reshape(1, D),
                        w2_2, b2_2.reshape(1, D), pooled[0], pooled[1],
                        wp1, bp1.reshape(1, D), wp2, bp2.reshape(1, D))
    return out
